# Initial kernel scaffold; baseline (speedup 1.0000x reference)
import jax
import jax.numpy as jnp
from jax.experimental import pallas as pl


def kernel(pos, z_indices, edge_index, batch, emb, W1, b1, W2, b2, Wfc, bfc):
    raise NotImplementedError("write your pallas kernel here")


# speedup vs baseline: 43.0327x; 43.0327x over previous
import functools

import jax
import jax.numpy as jnp
from jax import lax
from jax.experimental import pallas as pl
from jax.experimental.pallas import tpu as pltpu
from jax.experimental.pallas import tpu_sc as plsc

N = 100000
E = 6400000
NF = 8
H = 64
VOCAB = 5
NG = 1024
GHI = 32
GLO = 32

NC = 2
NS = 16
NW = NC * NS
SEG = 128
K = 8
CHUNK = K * SEG
NCHUNK = E // CHUNK
NSEGTOT = E // SEG
NPAD = 100096
ROWS_PER_TILE = NPAD // NS

BN = 2000
NBLK = N // BN


def _build_x_body(pos_ref, z_ref, emb_ref, x_ref):
    z = z_ref[0, 0, :]
    onehot = (z[:, None] == lax.broadcasted_iota(jnp.int32, (BN, VOCAB), 1))
    xe = jnp.dot(onehot.astype(jnp.float32), emb_ref[...],
                 preferred_element_type=jnp.float32)
    x_ref[...] = jnp.concatenate([pos_ref[...], xe], axis=1)


def _build_x(pos, z3, emb):
    return pl.pallas_call(
        _build_x_body,
        grid=(NBLK,),
        in_specs=[
            pl.BlockSpec((BN, 3), lambda i: (i, 0)),
            pl.BlockSpec((1, 1, BN), lambda i: (i, 0, 0)),
            pl.BlockSpec((VOCAB, VOCAB), lambda i: (0, 0)),
        ],
        out_specs=pl.BlockSpec((BN, NF), lambda i: (i, 0)),
        out_shape=jax.ShapeDtypeStruct((N, NF), jnp.float32),
    )(pos, z3, emb)


def _edge_agg_body(x_hbm, src_hbm, dst_hbm, zeros_hbm, out_hbm,
                   src_v, dst_v, rows_v, agg_sh, sem):
    c = lax.axis_index("c")
    s = lax.axis_index("s")
    wid = s * NC + c

    r0 = s * ROWS_PER_TILE
    pltpu.sync_copy(zeros_hbm.at[pl.ds(r0, ROWS_PER_TILE)],
                    agg_sh.at[pl.ds(r0, ROWS_PER_TILE)])
    plsc.subcore_barrier()

    base = NCHUNK // NW
    extra = NCHUNK - base * NW
    nloc = base + jnp.where(wid < extra, 1, 0)

    def body(i, carry):
        chunk = i * NW + wid
        pltpu.sync_copy(src_hbm.at[pl.ds(chunk * K, K)], src_v)
        pltpu.sync_copy(dst_hbm.at[pl.ds(chunk * K, K)], dst_v)
        cps = [pltpu.async_copy(x_hbm.at[src_v.at[j]], rows_v.at[j], sem)
               for j in range(K)]
        for cp in cps:
            cp.wait()
        for j in range(K):
            pltpu.sync_copy(rows_v.at[j], agg_sh.at[dst_v.at[j]], add=True)
        return carry

    lax.fori_loop(0, nloc, body, 0)

    plsc.subcore_barrier()
    pltpu.sync_copy(agg_sh.at[pl.ds(r0, ROWS_PER_TILE)],
                    out_hbm.at[c, pl.ds(r0, ROWS_PER_TILE)])


def _edge_agg(x, src3, dst3, zeros):
    mesh = plsc.VectorSubcoreMesh(core_axis_name="c", subcore_axis_name="s")
    fn = functools.partial(
        pl.kernel,
        out_type=jax.ShapeDtypeStruct((NC, NPAD, NF), jnp.float32),
        mesh=mesh,
        scratch_types=[
            pltpu.VMEM((K, SEG), jnp.int32),
            pltpu.VMEM((K, SEG), jnp.int32),
            pltpu.VMEM((K, SEG, NF), jnp.float32),
            pltpu.VMEM_SHARED((NPAD, NF), jnp.float32),
            pltpu.SemaphoreType.DMA,
        ],
        compiler_params=pltpu.CompilerParams(use_tc_tiling_on_sc=False),
    )(_edge_agg_body)
    return fn(x, src3, dst3, zeros)


def _mlp_pool_body(x_ref, p_ref, b3_ref, W1_ref, b1_ref, W2_ref, b2_ref,
                   Wfc_ref, bfc_ref, out_ref):
    i = pl.program_id(0)
    h = x_ref[...] + p_ref[0] + p_ref[1]
    a1 = jnp.maximum(
        jnp.dot(h, W1_ref[...], preferred_element_type=jnp.float32)
        + b1_ref[...], 0.0)
    a2 = jnp.maximum(
        jnp.dot(a1, W2_ref[...], preferred_element_type=jnp.float32)
        + b2_ref[...], 0.0)
    f = jnp.dot(a2, Wfc_ref[...], preferred_element_type=jnp.float32)

    seg = b3_ref[0, 0, :]
    lo = jnp.bitwise_and(seg, GLO - 1)
    hi = jnp.right_shift(seg, 5)
    oh_lo = (lo[:, None] == lax.broadcasted_iota(jnp.int32, (BN, GLO), 1))
    oh_hi_t = (hi[None, :] == lax.broadcasted_iota(jnp.int32, (GHI, BN), 0))
    contrib = jnp.dot(oh_hi_t.astype(jnp.float32),
                      f * oh_lo.astype(jnp.float32),
                      preferred_element_type=jnp.float32)

    @pl.when(i == 0)
    def _():
        out_ref[...] = contrib + bfc_ref[...]

    @pl.when(i != 0)
    def _():
        out_ref[...] += contrib


def _mlp_pool(x, parts, b3, W1, b1, W2, b2, Wfc, bfc):
    return pl.pallas_call(
        _mlp_pool_body,
        grid=(NBLK,),
        in_specs=[
            pl.BlockSpec((BN, NF), lambda i: (i, 0)),
            pl.BlockSpec((NC, BN, NF), lambda i: (0, i, 0)),
            pl.BlockSpec((1, 1, BN), lambda i: (i, 0, 0)),
            pl.BlockSpec((NF, H), lambda i: (0, 0)),
            pl.BlockSpec((1, H), lambda i: (0, 0)),
            pl.BlockSpec((H, H), lambda i: (0, 0)),
            pl.BlockSpec((1, H), lambda i: (0, 0)),
            pl.BlockSpec((H, 1), lambda i: (0, 0)),
            pl.BlockSpec((1, 1), lambda i: (0, 0)),
        ],
        out_specs=pl.BlockSpec((GHI, GLO), lambda i: (0, 0)),
        out_shape=jax.ShapeDtypeStruct((GHI, GLO), jnp.float32),
    )(x, parts, b3, W1, b1, W2, b2, Wfc, bfc)


def kernel(pos, z_indices, edge_index, batch, emb, W1, b1, W2, b2, Wfc, bfc):
    z3 = z_indices.astype(jnp.int32).reshape(NBLK, 1, BN)
    src3 = edge_index[0].astype(jnp.int32).reshape(NSEGTOT, SEG)
    dst3 = edge_index[1].astype(jnp.int32).reshape(NSEGTOT, SEG)
    b3 = batch.astype(jnp.int32).reshape(NBLK, 1, BN)
    zeros = jnp.zeros((NPAD, NF), jnp.float32)

    x = _build_x(pos, z3, emb)
    parts = _edge_agg(x, src3, dst3, zeros)
    pooled = _mlp_pool(x, parts, b3, W1.astype(jnp.float32),
                       b1.reshape(1, H), W2, b2.reshape(1, H),
                       Wfc, bfc.reshape(1, 1))
    return pooled.reshape(NG, 1)


# speedup vs baseline: 67.1575x; 1.5606x over previous
import functools

import jax
import jax.numpy as jnp
from jax import lax
from jax.experimental import pallas as pl
from jax.experimental.pallas import tpu as pltpu
from jax.experimental.pallas import tpu_sc as plsc

N = 100000
E = 6400000
NF = 8
H = 64
VOCAB = 5
NG = 1024
GHI = 32
GLO = 32

NC = 2
NS = 16
NW = NC * NS
SEG = 128
K = 8
CHUNK = K * SEG
NCHUNK = E // CHUNK
NSEGTOT = E // SEG
NPAD = 100096
ROWS_PER_TILE = NPAD // NS

BN = 2000
NBLK = N // BN


def _build_x_body(pos_ref, z_ref, emb_ref, x_ref):
    z = z_ref[0, 0, :]
    onehot = (z[:, None] == lax.broadcasted_iota(jnp.int32, (BN, VOCAB), 1))
    xe = jnp.dot(onehot.astype(jnp.float32), emb_ref[...],
                 preferred_element_type=jnp.float32)
    x_ref[...] = jnp.concatenate([pos_ref[...], xe], axis=1)


def _build_x(pos, z3, emb):
    return pl.pallas_call(
        _build_x_body,
        grid=(NBLK,),
        in_specs=[
            pl.BlockSpec((BN, 3), lambda i: (i, 0)),
            pl.BlockSpec((1, 1, BN), lambda i: (i, 0, 0)),
            pl.BlockSpec((VOCAB, VOCAB), lambda i: (0, 0)),
        ],
        out_specs=pl.BlockSpec((BN, NF), lambda i: (i, 0)),
        out_shape=jax.ShapeDtypeStruct((N, NF), jnp.float32),
    )(pos, z3, emb)


def _edge_agg_body(x_hbm, ei_hbm, zeros_hbm, out_hbm,
                   idx_v, rows_v, agg_sh, sem_i, sem_g, sem_s):
    c = lax.axis_index("c")
    s = lax.axis_index("s")
    wid = s * NC + c

    r0 = s * ROWS_PER_TILE
    pltpu.sync_copy(zeros_hbm.at[pl.ds(r0, ROWS_PER_TILE)],
                    agg_sh.at[pl.ds(r0, ROWS_PER_TILE)])
    plsc.subcore_barrier()

    base = NCHUNK // NW
    extra = NCHUNK - base * NW
    nloc = base + jnp.where(wid < extra, 1, 0)

    def idx_copy(i, slot):
        return pltpu.make_async_copy(
            ei_hbm.at[:, pl.ds((i * NW + wid) * K, K)], idx_v.at[slot], sem_i)

    def scat_copy(j, slot, b):
        return pltpu.make_async_copy(
            rows_v.at[b, j], agg_sh.at[idx_v.at[slot, 1, j]], sem_s)

    idx_copy(0, 0).start()

    def body(i, carry):
        b = jnp.bitwise_and(i, 1)
        s3 = lax.rem(i, 3)

        @pl.when(i + 1 < nloc)
        def _():
            idx_copy(i + 1, lax.rem(i + 1, 3)).start()

        idx_copy(i, s3).wait()
        gs = [pltpu.async_copy(x_hbm.at[idx_v.at[s3, 0, j]],
                               rows_v.at[b, j], sem_g) for j in range(K)]

        @pl.when(i > 0)
        def _():
            pb = jnp.bitwise_xor(b, 1)
            p3 = lax.rem(i + 2, 3)
            for j in range(K):
                scat_copy(j, p3, pb).wait()

        for cp in gs:
            cp.wait()
        for j in range(K):
            scat_copy(j, s3, b).start(add=True)
        return carry

    lax.fori_loop(0, nloc, body, 0)
    lb = jnp.bitwise_and(nloc - 1, 1)
    l3 = lax.rem(nloc - 1, 3)
    for j in range(K):
        scat_copy(j, l3, lb).wait()

    plsc.subcore_barrier()
    pltpu.sync_copy(agg_sh.at[pl.ds(r0, ROWS_PER_TILE)],
                    out_hbm.at[c, pl.ds(r0, ROWS_PER_TILE)])


def _edge_agg(x, ei3, zeros):
    mesh = plsc.VectorSubcoreMesh(core_axis_name="c", subcore_axis_name="s")
    fn = functools.partial(
        pl.kernel,
        out_type=jax.ShapeDtypeStruct((NC, NPAD, NF), jnp.float32),
        mesh=mesh,
        scratch_types=[
            pltpu.VMEM((3, 2, K, SEG), jnp.int32),
            pltpu.VMEM((2, K, SEG, NF), jnp.float32),
            pltpu.VMEM_SHARED((NPAD, NF), jnp.float32),
            pltpu.SemaphoreType.DMA,
            pltpu.SemaphoreType.DMA,
            pltpu.SemaphoreType.DMA,
        ],
        compiler_params=pltpu.CompilerParams(use_tc_tiling_on_sc=False),
    )(_edge_agg_body)
    return fn(x, ei3, zeros)


def _mlp_pool_body(x_ref, p_ref, b3_ref, W1_ref, b1_ref, W2_ref, b2_ref,
                   Wfc_ref, bfc_ref, out_ref):
    i = pl.program_id(0)
    h = x_ref[...] + p_ref[0] + p_ref[1]
    a1 = jnp.maximum(
        jnp.dot(h, W1_ref[...], preferred_element_type=jnp.float32)
        + b1_ref[...], 0.0)
    a2 = jnp.maximum(
        jnp.dot(a1, W2_ref[...], preferred_element_type=jnp.float32)
        + b2_ref[...], 0.0)
    f = jnp.dot(a2, Wfc_ref[...], preferred_element_type=jnp.float32)

    seg = b3_ref[0, 0, :]
    lo = jnp.bitwise_and(seg, GLO - 1)
    hi = jnp.right_shift(seg, 5)
    oh_lo = (lo[:, None] == lax.broadcasted_iota(jnp.int32, (BN, GLO), 1))
    oh_hi_t = (hi[None, :] == lax.broadcasted_iota(jnp.int32, (GHI, BN), 0))
    contrib = jnp.dot(oh_hi_t.astype(jnp.float32),
                      f * oh_lo.astype(jnp.float32),
                      preferred_element_type=jnp.float32)

    @pl.when(i == 0)
    def _():
        out_ref[...] = contrib + bfc_ref[...]

    @pl.when(i != 0)
    def _():
        out_ref[...] += contrib


def _mlp_pool(x, parts, b3, W1, b1, W2, b2, Wfc, bfc):
    return pl.pallas_call(
        _mlp_pool_body,
        grid=(NBLK,),
        in_specs=[
            pl.BlockSpec((BN, NF), lambda i: (i, 0)),
            pl.BlockSpec((NC, BN, NF), lambda i: (0, i, 0)),
            pl.BlockSpec((1, 1, BN), lambda i: (i, 0, 0)),
            pl.BlockSpec((NF, H), lambda i: (0, 0)),
            pl.BlockSpec((1, H), lambda i: (0, 0)),
            pl.BlockSpec((H, H), lambda i: (0, 0)),
            pl.BlockSpec((1, H), lambda i: (0, 0)),
            pl.BlockSpec((H, 1), lambda i: (0, 0)),
            pl.BlockSpec((1, 1), lambda i: (0, 0)),
        ],
        out_specs=pl.BlockSpec((GHI, GLO), lambda i: (0, 0)),
        out_shape=jax.ShapeDtypeStruct((GHI, GLO), jnp.float32),
    )(x, parts, b3, W1, b1, W2, b2, Wfc, bfc)


def kernel(pos, z_indices, edge_index, batch, emb, W1, b1, W2, b2, Wfc, bfc):
    z3 = z_indices.astype(jnp.int32).reshape(NBLK, 1, BN)
    ei3 = edge_index.astype(jnp.int32).reshape(2, NSEGTOT, SEG)
    b3 = batch.astype(jnp.int32).reshape(NBLK, 1, BN)
    zeros = jnp.zeros((NPAD, NF), jnp.float32)

    x = _build_x(pos, z3, emb)
    parts = _edge_agg(x, ei3, zeros)
    pooled = _mlp_pool(x, parts, b3, W1.astype(jnp.float32),
                       b1.reshape(1, H), W2, b2.reshape(1, H),
                       Wfc, bfc.reshape(1, 1))
    return pooled.reshape(NG, 1)
